# baseline (speedup 1.0000x reference)
import functools

import jax
import jax.numpy as jnp
from jax import lax
from jax.experimental import pallas as pl
from jax.experimental.pallas import tpu as pltpu
from jax.experimental.pallas import tpu_sc as plsc

NUM_STATE = 1000000
EMBED_DIM = 16
BATCH = 16384
FIELDS = 100

NC = 2
NS = 16
NW = NC * NS

B = BATCH * FIELDS
S = 128
K = 16
C = K * S
B_W = B // NW
N_CHUNK = B_W // C

assert B % NW == 0 and B_W % C == 0


def _gather_body(idx_hbm, table_hbm, out_hbm, idx_v, rows_v, sem):
  wid = lax.axis_index("s") * NC + lax.axis_index("c")
  base = wid * B_W

  def chunk(ci, _):
    off = pl.multiple_of(base + ci * C, C)
    row0 = pl.multiple_of(off // S, K)
    pltpu.sync_copy(idx_hbm.at[pl.ds(row0, K)], idx_v)
    copies = [
        pltpu.async_copy(
            table_hbm.at[idx_v.at[j]], rows_v.at[pl.ds(j * S, S)], sem
        )
        for j in range(K)
    ]
    for cp in copies:
      cp.wait()
    pltpu.sync_copy(rows_v, out_hbm.at[pl.ds(off, C)])
    return 0

  lax.fori_loop(0, N_CHUNK, chunk, 0)


@functools.partial(jax.jit, static_argnames=())
def _gather(idx2d, table):
  k = functools.partial(
      pl.kernel,
      out_type=jax.ShapeDtypeStruct((B, EMBED_DIM), jnp.float32),
      mesh=plsc.VectorSubcoreMesh(core_axis_name="c", subcore_axis_name="s"),
      scratch_types=[
          pltpu.VMEM((K, S), jnp.int32),
          pltpu.VMEM((C, EMBED_DIM), jnp.float32),
          pltpu.SemaphoreType.DMA,
      ],
      compiler_params=pltpu.CompilerParams(use_tc_tiling_on_sc=False),
  )(_gather_body)
  return k(idx2d, table)


def kernel(inputs, table):
  idx2d = inputs.reshape(B // S, S).astype(jnp.int32)
  out = _gather(idx2d, table)
  return out.reshape(BATCH, FIELDS, EMBED_DIM)


# speedup vs baseline: 1.0074x; 1.0074x over previous
import functools

import jax
import jax.numpy as jnp
from jax import lax
from jax.experimental import pallas as pl
from jax.experimental.pallas import tpu as pltpu
from jax.experimental.pallas import tpu_sc as plsc

NUM_STATE = 1000000
EMBED_DIM = 16
BATCH = 16384
FIELDS = 100

NC = 2
NS = 16
NW = NC * NS

B = BATCH * FIELDS
S = 128
K = 20
C = K * S
B_W = B // NW
N_CHUNK = B_W // C

assert B % NW == 0 and B_W % C == 0 and N_CHUNK % 2 == 0


def _gather_body(idx_hbm, table_hbm, out_hbm, idx_v, rows_v, isem, gsem, osem):
  wid = lax.axis_index("s") * NC + lax.axis_index("c")
  base = wid * B_W

  def idx_rows(ci):
    ci = jnp.minimum(ci, N_CHUNK - 1)
    return pl.multiple_of((base + ci * C) // S, K)

  def start_idx_load(ci, b):
    pltpu.async_copy(
        idx_hbm.at[pl.ds(idx_rows(ci), K)], idx_v.at[b], isem.at[b]
    )

  start_idx_load(0, 0)
  start_idx_load(1, 1)

  def chunk_pair(g, _):
    for b in range(2):
      ci = g * 2 + b
      off = pl.multiple_of(base + ci * C, C)
      pltpu.make_async_copy(
          idx_hbm.at[pl.ds(idx_rows(ci), K)], idx_v.at[b], isem.at[b]
      ).wait()
      copies = [
          pltpu.async_copy(
              table_hbm.at[idx_v.at[b].at[j]],
              rows_v.at[b].at[pl.ds(j * S, S)],
              gsem.at[b],
          )
          for j in range(K)
      ]
      for cp in copies:
        cp.wait()
      start_idx_load(ci + 2, b)

      @pl.when(g >= 1)
      def _():
        pltpu.make_async_copy(
            rows_v.at[b], out_hbm.at[pl.ds(off, C)], osem.at[b]
        ).wait()

      pltpu.async_copy(rows_v.at[b], out_hbm.at[pl.ds(off, C)], osem.at[b])
    return 0

  lax.fori_loop(0, N_CHUNK // 2, chunk_pair, 0)

  for b in range(2):
    off = pl.multiple_of(base + (N_CHUNK - 2 + b) * C, C)
    pltpu.make_async_copy(
        rows_v.at[b], out_hbm.at[pl.ds(off, C)], osem.at[b]
    ).wait()
    pltpu.make_async_copy(
        idx_hbm.at[pl.ds(idx_rows(N_CHUNK + b), K)], idx_v.at[b], isem.at[b]
    ).wait()


def _gather(idx2d, table):
  k = functools.partial(
      pl.kernel,
      out_type=jax.ShapeDtypeStruct((B, EMBED_DIM), jnp.float32),
      mesh=plsc.VectorSubcoreMesh(core_axis_name="c", subcore_axis_name="s"),
      scratch_types=[
          pltpu.VMEM((2, K, S), jnp.int32),
          pltpu.VMEM((2, C, EMBED_DIM), jnp.float32),
          pltpu.SemaphoreType.DMA((2,)),
          pltpu.SemaphoreType.DMA((2,)),
          pltpu.SemaphoreType.DMA((2,)),
      ],
      compiler_params=pltpu.CompilerParams(use_tc_tiling_on_sc=False),
  )(_gather_body)
  return k(idx2d, table)


def kernel(inputs, table):
  idx2d = inputs.reshape(B // S, S).astype(jnp.int32)
  out = _gather(idx2d, table)
  return out.reshape(BATCH, FIELDS, EMBED_DIM)


# speedup vs baseline: 2.4768x; 2.4587x over previous
import functools

import jax
import jax.numpy as jnp
from jax import lax
from jax.experimental import pallas as pl
from jax.experimental.pallas import tpu as pltpu
from jax.experimental.pallas import tpu_sc as plsc

NUM_STATE = 1000000
EMBED_DIM = 16
BATCH = 16384
FIELDS = 100

NC = 2
NS = 16
NW = NC * NS
ROWS_W = BATCH // NW
R = 16
N_CHUNK = ROWS_W // R

assert BATCH % NW == 0 and ROWS_W % R == 0 and N_CHUNK % 2 == 0


def _gather_body(idx_hbm, table_hbm, out_hbm, idx_v, rows_v, isem, gsem, osem):
  wid = lax.axis_index("s") * NC + lax.axis_index("c")
  base = wid * ROWS_W

  def row0(ci):
    ci = jnp.minimum(ci, N_CHUNK - 1)
    return pl.multiple_of(base + ci * R, R)

  def start_idx_load(ci, b):
    pltpu.async_copy(
        idx_hbm.at[pl.ds(row0(ci), R)], idx_v.at[b], isem.at[b]
    )

  start_idx_load(0, 0)
  start_idx_load(1, 1)

  def chunk_pair(g, _):
    for b in range(2):
      ci = g * 2 + b
      r0 = pl.multiple_of(base + ci * R, R)
      pltpu.make_async_copy(
          idx_hbm.at[pl.ds(row0(ci), R)], idx_v.at[b], isem.at[b]
      ).wait()
      copies = [
          pltpu.async_copy(
              table_hbm.at[idx_v.at[b].at[j]],
              rows_v.at[b].at[j],
              gsem.at[b],
          )
          for j in range(R)
      ]
      for cp in copies:
        cp.wait()
      start_idx_load(ci + 2, b)

      @pl.when(g >= 1)
      def _():
        pltpu.make_async_copy(
            rows_v.at[b], out_hbm.at[pl.ds(r0, R)], osem.at[b]
        ).wait()

      pltpu.async_copy(rows_v.at[b], out_hbm.at[pl.ds(r0, R)], osem.at[b])
    return 0

  lax.fori_loop(0, N_CHUNK // 2, chunk_pair, 0)

  for b in range(2):
    r0 = pl.multiple_of(base + (N_CHUNK - 2 + b) * R, R)
    pltpu.make_async_copy(
        rows_v.at[b], out_hbm.at[pl.ds(r0, R)], osem.at[b]
    ).wait()
    pltpu.make_async_copy(
        idx_hbm.at[pl.ds(row0(N_CHUNK + b), R)], idx_v.at[b], isem.at[b]
    ).wait()


def _gather(idx, table):
  k = functools.partial(
      pl.kernel,
      out_type=jax.ShapeDtypeStruct((BATCH, FIELDS, EMBED_DIM), jnp.float32),
      mesh=plsc.VectorSubcoreMesh(core_axis_name="c", subcore_axis_name="s"),
      scratch_types=[
          pltpu.VMEM((2, R, FIELDS), jnp.int32),
          pltpu.VMEM((2, R, FIELDS, EMBED_DIM), jnp.float32),
          pltpu.SemaphoreType.DMA((2,)),
          pltpu.SemaphoreType.DMA((2,)),
          pltpu.SemaphoreType.DMA((2,)),
      ],
      compiler_params=pltpu.CompilerParams(use_tc_tiling_on_sc=False),
  )(_gather_body)
  return k(idx, table)


def kernel(inputs, table):
  return _gather(inputs, table)


# speedup vs baseline: 3.9059x; 1.5770x over previous
import functools

import jax
import jax.numpy as jnp
from jax import lax
from jax.experimental import pallas as pl
from jax.experimental.pallas import tpu as pltpu
from jax.experimental.pallas import tpu_sc as plsc

NUM_STATE = 1000000
EMBED_DIM = 16
BATCH = 16384
FIELDS = 100

NC = 2
NS = 16
NW = NC * NS
BH = BATCH // 128
TPB = BH // NW
ROWS_W = TPB * 128
NT = FIELDS * TPB
EH = EMBED_DIM // 8

assert BATCH % (128 * NW) == 0 and NT % 2 == 0


def _iota16():
  return lax.iota(jnp.int32, 16)


def _gather_body(idx_hbm, table_hbm, out_hbm, slab, idxst, rows, trans,
                 gsem, osem):
  wid = lax.axis_index("s") * NC + lax.axis_index("c")
  b0 = wid * ROWS_W
  pltpu.sync_copy(idx_hbm.at[pl.ds(b0, ROWS_W)], slab)

  def tile_fb(t):
    t = jnp.minimum(t, NT - 1)
    return t // TPB, t % TPB

  def stage_col(t, b):
    f, bl = tile_fb(t)
    fv = jnp.full((16,), 0, jnp.int32) + f
    for k in range(8):
      rv = bl * 128 + k * 16 + _iota16()
      idxst[b, pl.ds(k * 16, 16)] = plsc.load_gather(slab, [rv, fv])

  def fire_gather(b):
    pltpu.async_copy(table_hbm.at[idxst.at[b]], rows.at[b], gsem.at[b])

  stage_col(0, 0)
  fire_gather(0)

  def pair(g, _):
    for b in range(2):
      t = g * 2 + b
      f, bl = tile_fb(t)
      bh = bl
      pltpu.make_async_copy(
          table_hbm.at[idxst.at[b]], rows.at[b], gsem.at[b]
      ).wait()
      stage_col(t + 1, 1 - b)
      fire_gather(1 - b)
      @pl.when(g >= 1)
      def _():
        for eh in range(EH):
          pltpu.make_async_copy(
              trans.at[b].at[eh], out_hbm.at[0, eh, 0], osem.at[b]
          ).wait()
      r2 = rows.at[b]
      for e in range(EMBED_DIM):
        ev = jnp.full((16,), e, jnp.int32)
        for k in range(8):
          rv = k * 16 + _iota16()
          trans[b, e // 8, e % 8, pl.ds(k * 16, 16)] = plsc.load_gather(
              r2, [rv, ev]
          )
      for eh in range(EH):
        pltpu.async_copy(
            trans.at[b].at[eh],
            out_hbm.at[f, eh, wid * TPB + bh],
            osem.at[b],
        )
    return 0

  lax.fori_loop(0, NT // 2, pair, 0)

  for b in range(2):
    for eh in range(EH):
      pltpu.make_async_copy(
          trans.at[b].at[eh], out_hbm.at[0, eh, 0], osem.at[b]
      ).wait()
  pltpu.make_async_copy(
      table_hbm.at[idxst.at[0]], rows.at[0], gsem.at[0]
  ).wait()


def _gather(idx, table):
  k = functools.partial(
      pl.kernel,
      out_type=jax.ShapeDtypeStruct((FIELDS, EH, BH, 8, 128), jnp.float32),
      mesh=plsc.VectorSubcoreMesh(core_axis_name="c", subcore_axis_name="s"),
      scratch_types=[
          pltpu.VMEM((ROWS_W, FIELDS), jnp.int32),
          pltpu.VMEM((2, 128), jnp.int32),
          pltpu.VMEM((2, 128, EMBED_DIM), jnp.float32),
          pltpu.VMEM((2, EH, 8, 128), jnp.float32),
          pltpu.SemaphoreType.DMA((2,)),
          pltpu.SemaphoreType.DMA((2,)),
      ],
      compiler_params=pltpu.CompilerParams(
          use_tc_tiling_on_sc=False, needs_layout_passes=False
      ),
  )(_gather_body)
  return k(idx, table)


def kernel(inputs, table):
  r5 = _gather(inputs, table)
  return r5.transpose(2, 4, 0, 1, 3).reshape(BATCH, FIELDS, EMBED_DIM)


# speedup vs baseline: 4.5015x; 1.1525x over previous
import functools

import jax
import jax.numpy as jnp
from jax import lax
from jax.experimental import pallas as pl
from jax.experimental.pallas import tpu as pltpu
from jax.experimental.pallas import tpu_sc as plsc

NUM_STATE = 1000000
EMBED_DIM = 16
BATCH = 16384
FIELDS = 100

NC = 2
NS = 16
NW = NC * NS
BH = BATCH // 128
TPB = BH // NW
ROWS_W = TPB * 128
NT = FIELDS * TPB
EH = EMBED_DIM // 8

assert BATCH % (128 * NW) == 0 and NT % 2 == 0


def _iota16():
  return lax.iota(jnp.int32, 16)


def _gather_body(idx_hbm, table_hbm, out_hbm, slab, idxst, rows, trans,
                 gsem, osem):
  wid = lax.axis_index("s") * NC + lax.axis_index("c")
  b0 = wid * ROWS_W
  pltpu.sync_copy(idx_hbm.at[pl.ds(b0, ROWS_W)], slab)

  def tile_fb(t):
    t = jnp.minimum(t, NT - 1)
    return t // TPB, t % TPB

  def stage_col(t, b):
    f, bl = tile_fb(t)
    fv = jnp.full((16,), 0, jnp.int32) + f
    vals = [
        plsc.load_gather(slab, [bl * 128 + k * 16 + _iota16(), fv])
        for k in range(8)
    ]
    for k in range(8):
      idxst[b, pl.ds(k * 16, 16)] = vals[k]

  def fire_gather(b):
    pltpu.async_copy(table_hbm.at[idxst.at[b]], rows.at[b], gsem.at[b])

  stage_col(0, 0)
  fire_gather(0)

  def pair(g, _):
    for b in range(2):
      t = g * 2 + b
      f, bl = tile_fb(t)
      bh = bl
      pltpu.make_async_copy(
          table_hbm.at[idxst.at[b]], rows.at[b], gsem.at[b]
      ).wait()
      stage_col(t + 1, 1 - b)
      fire_gather(1 - b)
      @pl.when(g >= 1)
      def _():
        for eh in range(EH):
          pltpu.make_async_copy(
              trans.at[b].at[eh], out_hbm.at[0, eh, 0], osem.at[b]
          ).wait()
      r2 = rows.at[b]
      for k in range(8):
        rv = k * 16 + _iota16()
        vals = [
            plsc.load_gather(r2, [rv, jnp.full((16,), e, jnp.int32)])
            for e in range(EMBED_DIM)
        ]
        for e in range(EMBED_DIM):
          trans[b, e // 8, e % 8, pl.ds(k * 16, 16)] = vals[e]
      for eh in range(EH):
        pltpu.async_copy(
            trans.at[b].at[eh],
            out_hbm.at[f, eh, wid * TPB + bh],
            osem.at[b],
        )
    return 0

  lax.fori_loop(0, NT // 2, pair, 0)

  for b in range(2):
    for eh in range(EH):
      pltpu.make_async_copy(
          trans.at[b].at[eh], out_hbm.at[0, eh, 0], osem.at[b]
      ).wait()
  pltpu.make_async_copy(
      table_hbm.at[idxst.at[0]], rows.at[0], gsem.at[0]
  ).wait()


def _gather(idx, table):
  k = functools.partial(
      pl.kernel,
      out_type=jax.ShapeDtypeStruct((FIELDS, EH, BH, 8, 128), jnp.float32),
      mesh=plsc.VectorSubcoreMesh(core_axis_name="c", subcore_axis_name="s"),
      scratch_types=[
          pltpu.VMEM((ROWS_W, FIELDS), jnp.int32),
          pltpu.VMEM((2, 128), jnp.int32),
          pltpu.VMEM((2, 128, EMBED_DIM), jnp.float32),
          pltpu.VMEM((2, EH, 8, 128), jnp.float32),
          pltpu.SemaphoreType.DMA((2,)),
          pltpu.SemaphoreType.DMA((2,)),
      ],
      compiler_params=pltpu.CompilerParams(
          use_tc_tiling_on_sc=False, needs_layout_passes=False
      ),
  )(_gather_body)
  return k(idx, table)


def kernel(inputs, table):
  r5 = _gather(inputs, table)
  return r5.transpose(2, 4, 0, 1, 3).reshape(BATCH, FIELDS, EMBED_DIM)
